# Initial kernel scaffold; baseline (speedup 1.0000x reference)
import jax
import jax.numpy as jnp
from jax.experimental import pallas as pl


def kernel(x, edge_index, edge_attr, alpha, mu, W1, b1, W2, b2, W3, b3, att_W, att_b, att_a, fc1_W, fc1_b, fc2_W, fc2_b):
    raise NotImplementedError("write your pallas kernel here")


# speedup vs baseline: 5.8304x; 5.8304x over previous
import functools

import jax
import jax.numpy as jnp
from jax import lax
from jax.experimental import pallas as pl
from jax.experimental.pallas import tpu as pltpu
from jax.experimental.pallas import tpu_sc as plsc

N_NODES = 10000
N_EDGES = 320000
D = 128
NC = 2
NS = 16
NW = NC * NS
EPW = N_EDGES // NW
CH = 80
NCHUNK = EPW // CH
RPS = N_NODES // NS

_MESH = dict(core_axis_name="c", subcore_axis_name="s", num_cores=NC,
             num_subcores=NS)
_SC_PARAMS = pltpu.CompilerParams(needs_layout_passes=False,
                                  use_tc_tiling_on_sc=False)
F32 = jnp.float32
I32 = jnp.int32


def _wid():
    return lax.axis_index("s") * NC + lax.axis_index("c")


def _lanes():
    return lax.iota(I32, 16)


def _rmw1(ref, pos, fn):
    lanes = _lanes()
    mk = lanes < 1
    idx = jnp.where(mk, jnp.zeros((16,), I32) + pos, 0)
    cur = plsc.load_gather(ref, [idx], mask=mk)
    plsc.store_scatter(ref, [idx], fn(cur), mask=mk)


def _store1(ref, pos, val):
    lanes = _lanes()
    mk = lanes < 1
    idx = jnp.where(mk, jnp.zeros((16,), I32) + pos, 0)
    plsc.store_scatter(ref, [idx], jnp.zeros((16,), F32) + val, mask=mk)


@functools.partial(
    pl.kernel,
    out_type=jax.ShapeDtypeStruct((NW, N_NODES), F32),
    mesh=plsc.VectorSubcoreMesh(**_MESH),
    compiler_params=_SC_PARAMS,
    scratch_types=[
        pltpu.VMEM((NCHUNK, CH), I32),
        pltpu.VMEM((N_NODES,), F32),
    ],
)
def _sc_degree(dst3, zeros_n, out, idx_d, cnt):
    wid = _wid()
    pltpu.sync_copy(zeros_n, cnt)
    pltpu.sync_copy(dst3.at[wid], idx_d)

    @pl.loop(0, NCHUNK)
    def _chunk(j):
        for g in range(CH // 16):
            dv = idx_d[j, pl.ds(g * 16, 16)]
            for l in range(16):
                _rmw1(cnt, dv[l], lambda c: c + 1.0)

    pltpu.sync_copy(cnt, out.at[wid])


@functools.partial(
    pl.kernel,
    out_type=jax.ShapeDtypeStruct((NC, N_NODES, D), F32),
    mesh=plsc.VectorSubcoreMesh(**_MESH),
    compiler_params=_SC_PARAMS,
    scratch_types=[
        pltpu.VMEM((NCHUNK, CH), I32),
        pltpu.VMEM((NCHUNK, CH), I32),
        pltpu.VMEM((CH, D), F32),
        pltpu.SemaphoreType.DMA,
        pltpu.VMEM_SHARED((N_NODES, D), F32),
    ],
)
def _sc_gcn(xws, src3, dst3, zeros128, out, idx_s, idx_d, rows, sem, acc):
    cid = lax.axis_index("c")
    sid = lax.axis_index("s")
    wid = sid * NC + cid
    pltpu.sync_copy(zeros128.at[pl.ds(sid * RPS, RPS)],
                    acc.at[pl.ds(sid * RPS, RPS)])
    pltpu.sync_copy(src3.at[wid], idx_s)
    pltpu.sync_copy(dst3.at[wid], idx_d)
    plsc.subcore_barrier()

    @pl.loop(0, NCHUNK)
    def _chunk(j):
        pltpu.async_copy(xws.at[idx_s.at[j]], rows, sem).wait()
        pltpu.sync_copy(rows, acc.at[idx_d.at[j]], add=True)

    plsc.subcore_barrier()
    pltpu.sync_copy(acc.at[pl.ds(sid * RPS, RPS)],
                    out.at[cid, pl.ds(sid * RPS, RPS)])


@functools.partial(
    pl.kernel,
    out_type=(
        jax.ShapeDtypeStruct((NW, NCHUNK, 4 * CH), F32),
        jax.ShapeDtypeStruct((NW, 4 * N_NODES), F32),
    ),
    mesh=plsc.VectorSubcoreMesh(**_MESH),
    compiler_params=_SC_PARAMS,
    scratch_types=[
        pltpu.VMEM((NCHUNK, CH), I32),
        pltpu.VMEM((NCHUNK, CH), I32),
        pltpu.VMEM((CH, 64), F32),
        pltpu.VMEM((CH, 64), F32),
        pltpu.VMEM((CH,), F32),
        pltpu.VMEM((4 * CH + 16,), F32),
        pltpu.VMEM((4 * N_NODES,), F32),
        pltpu.VMEM((64,), F32),
        pltpu.VMEM((64,), F32),
        pltpu.SemaphoreType.DMA,
    ],
)
def _sc_att1(pa, qa, src3, dst3, attr, aw0, aa, neg4, scores_o, mpart_o,
             idx_s, idx_d, pa_buf, qa_buf, attr_v, sc_buf, m_loc, aw0_v,
             aa_v, sem):
    wid = _wid()
    pltpu.sync_copy(neg4, m_loc)
    pltpu.sync_copy(src3.at[wid], idx_s)
    pltpu.sync_copy(dst3.at[wid], idx_d)
    pltpu.sync_copy(aw0, aw0_v)
    pltpu.sync_copy(aa, aa_v)
    lanes = _lanes()
    aw0_h = [aw0_v[pl.ds(h * 16, 16)] for h in range(4)]
    aa_h = [aa_v[pl.ds(h * 16, 16)] for h in range(4)]

    @pl.loop(0, NCHUNK)
    def _chunk(j):
        pltpu.async_copy(pa.at[idx_s.at[j]], pa_buf, sem).wait()
        pltpu.async_copy(qa.at[idx_d.at[j]], qa_buf, sem).wait()
        pltpu.sync_copy(attr.at[pl.ds(wid * EPW + j * CH, CH)], attr_v)

        @pl.loop(0, CH // 16)
        def _group(g):
            attr_g = attr_v[pl.ds(g * 16, 16)]
            rowi = g * 16 + lanes
            for h in range(4):
                accv = jnp.zeros((16,), F32)
                for dd in range(16):
                    colv = jnp.zeros((16,), I32) + (h * 16 + dd)
                    pav = plsc.load_gather(pa_buf, [rowi, colv])
                    qav = plsc.load_gather(qa_buf, [rowi, colv])
                    t = jnp.maximum(pav + qav + attr_g * aw0_h[h][dd], 0.0)
                    accv = accv + t * aa_h[h][dd]
                plsc.store_scatter(sc_buf, [rowi * 4 + h], accv * 10.0)
            srcv = idx_s[j, pl.ds(g * 16, 16)]
            mk = lanes < 4
            off = jnp.where(mk, lanes, 0)
            for l in range(16):
                srce = srcv[l]
                v4 = sc_buf[pl.ds((g * 16 + l) * 4, 16)]
                midx = jnp.where(mk, srce * 4 + off, 0)
                cur = plsc.load_gather(m_loc, [midx], mask=mk)
                plsc.store_scatter(m_loc, [midx], jnp.maximum(cur, v4),
                                   mask=mk)

        pltpu.sync_copy(sc_buf.at[pl.ds(0, 4 * CH)], scores_o.at[wid, j])

    pltpu.sync_copy(m_loc, mpart_o.at[wid])


@functools.partial(
    pl.kernel,
    out_type=(
        jax.ShapeDtypeStruct((NW, NCHUNK, 4 * CH), F32),
        jax.ShapeDtypeStruct((NW, 4 * N_NODES), F32),
    ),
    mesh=plsc.VectorSubcoreMesh(**_MESH),
    compiler_params=_SC_PARAMS,
    scratch_types=[
        pltpu.VMEM((NCHUNK, CH), I32),
        pltpu.VMEM((4 * CH + 16,), F32),
        pltpu.VMEM((4 * CH + 16,), F32),
        pltpu.VMEM((4 * N_NODES,), F32),
        pltpu.VMEM((4 * N_NODES,), F32),
    ],
)
def _sc_att2(scores3, src3, mtab, zeros4, e_o, spart_o,
             idx_s, sc_buf, e_vbuf, m_tab, s_loc):
    wid = _wid()
    pltpu.sync_copy(mtab, m_tab)
    pltpu.sync_copy(zeros4, s_loc)
    pltpu.sync_copy(src3.at[wid], idx_s)
    lanes = _lanes()

    @pl.loop(0, NCHUNK)
    def _chunk(j):
        pltpu.sync_copy(scores3.at[wid, j], sc_buf.at[pl.ds(0, 4 * CH)])

        @pl.loop(0, CH // 16)
        def _group(g):
            srcv = idx_s[j, pl.ds(g * 16, 16)]
            rowi4 = (g * 16 + lanes) * 4
            for h in range(4):
                scv = plsc.load_gather(sc_buf, [rowi4 + h])
                mg = plsc.load_gather(m_tab, [srcv * 4 + h])
                plsc.store_scatter(e_vbuf, [rowi4 + h], jnp.exp(scv - mg))
            mk = lanes < 4
            off = jnp.where(mk, lanes, 0)
            for l in range(16):
                srce = srcv[l]
                e4 = e_vbuf[pl.ds((g * 16 + l) * 4, 16)]
                sidx = jnp.where(mk, srce * 4 + off, 0)
                cur = plsc.load_gather(s_loc, [sidx], mask=mk)
                plsc.store_scatter(s_loc, [sidx], cur + e4, mask=mk)

        pltpu.sync_copy(e_vbuf.at[pl.ds(0, 4 * CH)], e_o.at[wid, j])

    pltpu.sync_copy(s_loc, spart_o.at[wid])


@functools.partial(
    pl.kernel,
    out_type=jax.ShapeDtypeStruct((N_EDGES,), F32),
    mesh=plsc.VectorSubcoreMesh(**_MESH),
    compiler_params=_SC_PARAMS,
    scratch_types=[
        pltpu.VMEM((NCHUNK, CH), I32),
        pltpu.VMEM((NCHUNK, CH), I32),
        pltpu.VMEM((CH, 256), F32),
        pltpu.VMEM((CH, 256), F32),
        pltpu.VMEM((4 * CH,), F32),
        pltpu.VMEM((CH,), F32),
        pltpu.VMEM((CH,), F32),
        pltpu.VMEM((CH,), F32),
        pltpu.VMEM((4 * N_NODES,), F32),
        pltpu.VMEM((256,), F32),
        pltpu.VMEM((256,), F32),
        pltpu.VMEM((256,), F32),
        pltpu.VMEM((16,), F32),
        pltpu.SemaphoreType.DMA,
        pltpu.SemaphoreType.DMA,
    ],
)
def _sc_att3(pf, qf, e3, src3, dst3, attr, recip, fr, cvec, w2, b2, z_o,
             idx_s, idx_d, pf_buf, qf_buf, e_vbuf, attr_v, att_buf, z_buf,
             recip_v, fr_v, cv_v, w2_v, b2_v, sem, sem2):
    wid = _wid()
    pltpu.sync_copy(recip, recip_v)
    pltpu.sync_copy(fr, fr_v)
    pltpu.sync_copy(cvec, cv_v)
    pltpu.sync_copy(w2, w2_v)
    pltpu.sync_copy(b2, b2_v)
    pltpu.sync_copy(src3.at[wid], idx_s)
    pltpu.sync_copy(dst3.at[wid], idx_d)
    fr_s = [fr_v[pl.ds(jj * 16, 16)] for jj in range(16)]
    cv_s = [cv_v[pl.ds(jj * 16, 16)] for jj in range(16)]
    w2_s = [w2_v[pl.ds(jj * 16, 16)] for jj in range(16)]
    b2_s = b2_v[pl.ds(0, 16)][0]
    lanes = _lanes()

    @pl.loop(0, NCHUNK)
    def _chunk(j):
        cp1 = pltpu.async_copy(pf.at[idx_s.at[j]], pf_buf, sem)
        cp2 = pltpu.async_copy(qf.at[idx_d.at[j]], qf_buf, sem2)
        pltpu.sync_copy(e3.at[wid, j], e_vbuf)
        pltpu.sync_copy(attr.at[pl.ds(wid * EPW + j * CH, CH)], attr_v)
        cp1.wait()
        cp2.wait()

        @pl.loop(0, CH // 16)
        def _group(g):
            srcv = idx_s[j, pl.ds(g * 16, 16)]
            rowi4 = (g * 16 + lanes) * 4
            att_g = jnp.zeros((16,), F32)
            for h in range(4):
                rv = plsc.load_gather(recip_v, [srcv * 4 + h])
                ev = plsc.load_gather(e_vbuf, [rowi4 + h])
                att_g = att_g + ev * rv
            attr_g = attr_v[pl.ds(g * 16, 16)]
            for l in range(16):
                e = g * 16 + l
                atte = att_g[l]
                attre = attr_g[l]
                acc = jnp.zeros((16,), F32)
                for jj in range(16):
                    pfv = pf_buf[e, pl.ds(jj * 16, 16)]
                    qfv = qf_buf[e, pl.ds(jj * 16, 16)]
                    hv = jnp.maximum(
                        atte * (pfv + qfv) + attre * fr_s[jj] + cv_s[jj], 0.0)
                    acc = acc + hv * w2_s[jj]
                _store1(z_buf, e, jnp.sum(acc) + b2_s)

        pltpu.sync_copy(z_buf, z_o.at[pl.ds(wid * EPW + j * CH, CH)])


def _dis_col(cnt_t):
    return lax.rsqrt(jnp.sum(cnt_t, axis=1, keepdims=True) + 1.0)


def _tc_first(cnt_t_ref, x_ref, w_ref, out_ref):
    dis = _dis_col(cnt_t_ref[...])
    xw = jnp.dot(x_ref[...], w_ref[...], preferred_element_type=F32)
    out_ref[...] = xw * dis


def _tc_layer(cnt_t_ref, parts_ref, xws_ref, b_ref, w_ref, out_ref):
    dis = _dis_col(cnt_t_ref[...])
    acc = parts_ref[0] + parts_ref[1] + xws_ref[...]
    h = jnp.maximum(acc * dis + b_ref[...][None, :], 0.0)
    out_ref[...] = jnp.dot(h, w_ref[...], preferred_element_type=F32) * dis


def _tc_head(cnt_t_ref, parts_ref, xws_ref, b_ref, wa1_ref, ab_ref, wa2_ref,
             f1_ref, f2_ref, pa_ref, qa_ref, pf_ref, qf_ref):
    dis = _dis_col(cnt_t_ref[...])
    acc = parts_ref[0] + parts_ref[1] + xws_ref[...]
    h = jnp.maximum(acc * dis + b_ref[...][None, :], 0.0)
    pa_ref[...] = (jnp.dot(h, wa1_ref[...], preferred_element_type=F32)
                   + ab_ref[...][None, :])
    qa_ref[...] = jnp.dot(h, wa2_ref[...], preferred_element_type=F32)
    pf_ref[...] = jnp.dot(h, f1_ref[...], preferred_element_type=F32)
    qf_ref[...] = jnp.dot(h, f2_ref[...], preferred_element_type=F32)


def _tc_maxred(parts_ref, out_ref):
    out_ref[...] = jnp.max(parts_ref[...], axis=0, keepdims=True)


def _tc_recip(parts_ref, alpha_ref, out_ref):
    s = jnp.sum(parts_ref[...], axis=0, keepdims=True)
    out_ref[...] = (1.0 + alpha_ref[0, 0]) * 0.25 / (s + 1e-16)


def _tc_prob(z_ref, mu_ref, out_ref):
    z = z_ref[...]
    sp = jnp.maximum(z, 0.0) + jnp.log1p(jnp.exp(-jnp.abs(z)))
    out_ref[...] = jnp.clip(sp / mu_ref[0, 0], 0.0, 1.0)


def _vspec():
    return pl.BlockSpec(memory_space=pltpu.ANY)


def kernel(x, edge_index, edge_attr, alpha, mu, W1, b1, W2, b2, W3, b3,
           att_W, att_b, att_a, fc1_W, fc1_b, fc2_W, fc2_b):
    f32 = jnp.float32
    src3 = edge_index[0].reshape(NW, NCHUNK, CH)
    dst3 = edge_index[1].reshape(NW, NCHUNK, CH)
    attr = edge_attr.reshape(-1)

    zeros_n = jnp.zeros((N_NODES,), f32)
    zeros128 = jnp.zeros((N_NODES, D), f32)
    zeros4 = jnp.zeros((4 * N_NODES,), f32)
    neg4 = jnp.full((4 * N_NODES,), -1e30, f32)

    cntp = _sc_degree(dst3, zeros_n)
    cnt_t = cntp.T

    xw1s = pl.pallas_call(
        _tc_first,
        out_shape=jax.ShapeDtypeStruct((N_NODES, D), f32),
    )(cnt_t, x, W1)
    p1 = _sc_gcn(xw1s, src3, dst3, zeros128)

    xw2s = pl.pallas_call(
        _tc_layer,
        out_shape=jax.ShapeDtypeStruct((N_NODES, D), f32),
    )(cnt_t, p1, xw1s, b1, W2)
    p2 = _sc_gcn(xw2s, src3, dst3, zeros128)

    xw3s = pl.pallas_call(
        _tc_layer,
        out_shape=jax.ShapeDtypeStruct((N_NODES, D), f32),
    )(cnt_t, p2, xw2s, b2, W3)
    p3 = _sc_gcn(xw3s, src3, dst3, zeros128)

    pa, qa, pf, qf = pl.pallas_call(
        _tc_head,
        out_shape=(
            jax.ShapeDtypeStruct((N_NODES, 64), f32),
            jax.ShapeDtypeStruct((N_NODES, 64), f32),
            jax.ShapeDtypeStruct((N_NODES, 256), f32),
            jax.ShapeDtypeStruct((N_NODES, 256), f32),
        ),
    )(cnt_t, p3, xw3s, b3, att_W[:D], att_b, att_W[D:2 * D],
      fc1_W[:D], fc1_W[D:2 * D])

    aw0 = att_W[2 * D] * 20.0
    aa = att_a.reshape(-1)

    scores3, mpart = _sc_att1(pa, qa, src3, dst3, attr, aw0, aa, neg4)
    mtab = pl.pallas_call(
        _tc_maxred,
        out_shape=jax.ShapeDtypeStruct((1, 4 * N_NODES), f32),
    )(mpart).reshape(-1)

    e3, spart = _sc_att2(scores3, src3, mtab, zeros4)
    recip = pl.pallas_call(
        _tc_recip,
        out_shape=jax.ShapeDtypeStruct((1, 4 * N_NODES), f32),
        in_specs=[pl.BlockSpec(memory_space=pltpu.VMEM),
                  pl.BlockSpec(memory_space=pltpu.SMEM)],
    )(spart, alpha).reshape(-1)

    fr = fc1_W[2 * D]
    cvec = alpha[0, 0] * fc1_W[2 * D + 1] + mu[0, 0] * fc1_W[2 * D + 2] + fc1_b
    w2 = fc2_W.reshape(-1)
    b2 = jnp.pad(fc2_b, (0, 15))
    z = _sc_att3(pf, qf, e3, src3, dst3, attr, recip, fr, cvec, w2, b2)

    prob = pl.pallas_call(
        _tc_prob,
        out_shape=jax.ShapeDtypeStruct((N_EDGES // D, D), f32),
        in_specs=[pl.BlockSpec(memory_space=pltpu.VMEM),
                  pl.BlockSpec(memory_space=pltpu.SMEM)],
    )(z.reshape(N_EDGES // D, D), mu)
    return prob.reshape(N_EDGES, 1)


# speedup vs baseline: 7.8095x; 1.3394x over previous
import functools

import jax
import jax.numpy as jnp
from jax import lax
from jax.experimental import pallas as pl
from jax.experimental.pallas import tpu as pltpu
from jax.experimental.pallas import tpu_sc as plsc

N_NODES = 10000
N_EDGES = 320000
D = 128
NC = 2
NS = 16
NW = NC * NS
EPW = N_EDGES // NW
CH = 80
NCHUNK = EPW // CH
RPS = N_NODES // NS

_MESH = dict(core_axis_name="c", subcore_axis_name="s", num_cores=NC,
             num_subcores=NS)
_SC_PARAMS = pltpu.CompilerParams(needs_layout_passes=False,
                                  use_tc_tiling_on_sc=False)
F32 = jnp.float32
I32 = jnp.int32


def _wid():
    return lax.axis_index("s") * NC + lax.axis_index("c")


def _lanes():
    return lax.iota(I32, 16)


def _rmw1(ref, pos, fn):
    lanes = _lanes()
    mk = lanes < 1
    idx = jnp.where(mk, jnp.zeros((16,), I32) + pos, 0)
    cur = plsc.load_gather(ref, [idx], mask=mk)
    plsc.store_scatter(ref, [idx], fn(cur), mask=mk)


def _store1(ref, pos, val):
    lanes = _lanes()
    mk = lanes < 1
    idx = jnp.where(mk, jnp.zeros((16,), I32) + pos, 0)
    plsc.store_scatter(ref, [idx], jnp.zeros((16,), F32) + val, mask=mk)


@functools.partial(
    pl.kernel,
    out_type=jax.ShapeDtypeStruct((NW, N_NODES), F32),
    mesh=plsc.VectorSubcoreMesh(**_MESH),
    compiler_params=_SC_PARAMS,
    scratch_types=[
        pltpu.VMEM((NCHUNK, CH), I32),
        pltpu.VMEM((N_NODES,), F32),
    ],
)
def _sc_degree(dst3, zeros_n, out, idx_d, cnt):
    wid = _wid()
    pltpu.sync_copy(zeros_n, cnt)
    pltpu.sync_copy(dst3.at[wid], idx_d)

    @pl.loop(0, NCHUNK)
    def _chunk(j):
        for g in range(CH // 16):
            dv = idx_d[j, pl.ds(g * 16, 16)]
            for l in range(16):
                _rmw1(cnt, dv[l], lambda c: c + 1.0)

    pltpu.sync_copy(cnt, out.at[wid])


@functools.partial(
    pl.kernel,
    out_type=jax.ShapeDtypeStruct((NC, N_NODES, D), F32),
    mesh=plsc.VectorSubcoreMesh(**_MESH),
    compiler_params=_SC_PARAMS,
    scratch_types=[
        pltpu.VMEM((NCHUNK, CH), I32),
        pltpu.VMEM((NCHUNK, CH), I32),
        pltpu.VMEM((CH, D), F32),
        pltpu.VMEM((CH, D), F32),
        pltpu.SemaphoreType.DMA,
        pltpu.SemaphoreType.DMA,
        pltpu.SemaphoreType.DMA,
        pltpu.SemaphoreType.DMA,
        pltpu.VMEM_SHARED((N_NODES, D), F32),
    ],
)
def _sc_gcn(xws, src3, dst3, zeros128, out, idx_s, idx_d, rows_a, rows_b,
            ga, gb, sa, sb, acc):
    cid = lax.axis_index("c")
    sid = lax.axis_index("s")
    wid = sid * NC + cid
    pltpu.sync_copy(zeros128.at[pl.ds(sid * RPS, RPS)],
                    acc.at[pl.ds(sid * RPS, RPS)])
    pltpu.sync_copy(src3.at[wid], idx_s)
    pltpu.sync_copy(dst3.at[wid], idx_d)
    plsc.subcore_barrier()

    def fg(j, rows, sem):
        pltpu.async_copy(xws.at[idx_s.at[j]], rows, sem)

    def fs(j, rows, sem):
        pltpu.async_copy(rows, acc.at[idx_d.at[j]], sem, add=True)

    def wg(rows, sem):
        pltpu.make_async_copy(xws.at[idx_s.at[0]], rows, sem).wait()

    def ws(rows, sem):
        pltpu.make_async_copy(rows, acc.at[idx_d.at[0]], sem).wait()

    fg(0, rows_a, ga)

    @pl.loop(0, (NCHUNK - 1) // 2)
    def _pair(k):
        j = k * 2
        wg(rows_a, ga)
        fs(j, rows_a, sa)
        fg(j + 1, rows_b, gb)
        wg(rows_b, gb)
        fs(j + 1, rows_b, sb)
        ws(rows_a, sa)
        fg(j + 2, rows_a, ga)
        ws(rows_b, sb)

    wg(rows_a, ga)
    fs(NCHUNK - 1, rows_a, sa)
    ws(rows_a, sa)
    plsc.subcore_barrier()
    pltpu.sync_copy(acc.at[pl.ds(sid * RPS, RPS)],
                    out.at[cid, pl.ds(sid * RPS, RPS)])


@functools.partial(
    pl.kernel,
    out_type=(
        jax.ShapeDtypeStruct((NW, NCHUNK, 4 * CH), F32),
        jax.ShapeDtypeStruct((NW, 4 * N_NODES), F32),
    ),
    mesh=plsc.VectorSubcoreMesh(**_MESH),
    compiler_params=_SC_PARAMS,
    scratch_types=[
        pltpu.VMEM((NCHUNK, CH), I32),
        pltpu.VMEM((NCHUNK, CH), I32),
        pltpu.VMEM((CH, 64), F32),
        pltpu.VMEM((CH, 64), F32),
        pltpu.VMEM((CH, 64), F32),
        pltpu.VMEM((CH, 64), F32),
        pltpu.VMEM((EPW,), F32),
        pltpu.VMEM((4 * CH + 16,), F32),
        pltpu.VMEM((4 * CH + 16,), F32),
        pltpu.VMEM((4 * N_NODES,), F32),
        pltpu.VMEM((64,), F32),
        pltpu.VMEM((64,), F32),
        pltpu.SemaphoreType.DMA,
        pltpu.SemaphoreType.DMA,
        pltpu.SemaphoreType.DMA,
        pltpu.SemaphoreType.DMA,
    ],
)
def _sc_att1(pa, qa, src3, dst3, attr, aw0, aa, neg4, scores_o, mpart_o,
             idx_s, idx_d, pa_a, qa_a, pa_b, qa_b, attr_all, sc_a, sc_b,
             m_loc, aw0_v, aa_v, ga, gb, oa, ob):
    wid = _wid()
    pltpu.sync_copy(neg4, m_loc)
    pltpu.sync_copy(src3.at[wid], idx_s)
    pltpu.sync_copy(dst3.at[wid], idx_d)
    pltpu.sync_copy(attr.at[pl.ds(wid * EPW, EPW)], attr_all)
    pltpu.sync_copy(aw0, aw0_v)
    pltpu.sync_copy(aa, aa_v)
    lanes = _lanes()
    aw0_h = [aw0_v[pl.ds(h * 16, 16)] for h in range(4)]
    aa_h = [aa_v[pl.ds(h * 16, 16)] for h in range(4)]

    def fg(j, pa_x, qa_x, sem):
        pltpu.async_copy(pa.at[idx_s.at[j]], pa_x, sem)
        pltpu.async_copy(qa.at[idx_d.at[j]], qa_x, sem)

    def wg(pa_x, qa_x, sem):
        pltpu.make_async_copy(pa.at[idx_s.at[0]], pa_x, sem).wait()
        pltpu.make_async_copy(qa.at[idx_d.at[0]], qa_x, sem).wait()

    def compute(j, pa_x, qa_x, sc_x, osem, guard):
        def _drain():
            pltpu.make_async_copy(sc_x.at[pl.ds(0, 4 * CH)],
                                  scores_o.at[wid, 0], osem).wait()

        if guard is True:
            _drain()
        else:
            pl.when(guard)(_drain)

        @pl.loop(0, CH // 16)
        def _group(g):
            attr_g = attr_all[pl.ds(j * CH + g * 16, 16)]
            rowi = g * 16 + lanes
            for h in range(4):
                accv = jnp.zeros((16,), F32)
                for dd in range(16):
                    colv = jnp.zeros((16,), I32) + (h * 16 + dd)
                    pav = plsc.load_gather(pa_x, [rowi, colv])
                    qav = plsc.load_gather(qa_x, [rowi, colv])
                    t = jnp.maximum(pav + qav + attr_g * aw0_h[h][dd], 0.0)
                    accv = accv + t * aa_h[h][dd]
                plsc.store_scatter(sc_x, [rowi * 4 + h], accv * 10.0)
            srcv = idx_s[j, pl.ds(g * 16, 16)]
            mk = lanes < 4
            off = jnp.where(mk, lanes, 0)
            for l in range(16):
                srce = srcv[l]
                v4 = sc_x[pl.ds((g * 16 + l) * 4, 16)]
                midx = jnp.where(mk, srce * 4 + off, 0)
                cur = plsc.load_gather(m_loc, [midx], mask=mk)
                plsc.store_scatter(m_loc, [midx], jnp.maximum(cur, v4),
                                   mask=mk)

        pltpu.async_copy(sc_x.at[pl.ds(0, 4 * CH)], scores_o.at[wid, j],
                         osem)

    fg(0, pa_a, qa_a, ga)

    @pl.loop(0, (NCHUNK - 1) // 2)
    def _pair(k):
        j = k * 2
        wg(pa_a, qa_a, ga)
        fg(j + 1, pa_b, qa_b, gb)
        compute(j, pa_a, qa_a, sc_a, oa, k > 0)
        wg(pa_b, qa_b, gb)
        fg(j + 2, pa_a, qa_a, ga)
        compute(j + 1, pa_b, qa_b, sc_b, ob, k > 0)

    wg(pa_a, qa_a, ga)
    compute(NCHUNK - 1, pa_a, qa_a, sc_a, oa, True)
    pltpu.make_async_copy(sc_a.at[pl.ds(0, 4 * CH)], scores_o.at[wid, 0],
                          oa).wait()
    pltpu.make_async_copy(sc_b.at[pl.ds(0, 4 * CH)], scores_o.at[wid, 0],
                          ob).wait()
    pltpu.sync_copy(m_loc, mpart_o.at[wid])


@functools.partial(
    pl.kernel,
    out_type=(
        jax.ShapeDtypeStruct((NW, NCHUNK, 4 * CH), F32),
        jax.ShapeDtypeStruct((NW, 4 * N_NODES), F32),
    ),
    mesh=plsc.VectorSubcoreMesh(**_MESH),
    compiler_params=_SC_PARAMS,
    scratch_types=[
        pltpu.VMEM((NCHUNK, CH), I32),
        pltpu.VMEM((4 * CH + 16,), F32),
        pltpu.VMEM((4 * CH + 16,), F32),
        pltpu.VMEM((4 * CH + 16,), F32),
        pltpu.VMEM((4 * CH + 16,), F32),
        pltpu.VMEM((4 * N_NODES,), F32),
        pltpu.VMEM((4 * N_NODES,), F32),
        pltpu.SemaphoreType.DMA,
        pltpu.SemaphoreType.DMA,
        pltpu.SemaphoreType.DMA,
        pltpu.SemaphoreType.DMA,
    ],
)
def _sc_att2(scores3, src3, mtab, zeros4, e_o, spart_o,
             idx_s, sc_a, sc_b, e_va, e_vb, m_tab, s_loc, ga, gb, oa, ob):
    wid = _wid()
    pltpu.sync_copy(mtab, m_tab)
    pltpu.sync_copy(zeros4, s_loc)
    pltpu.sync_copy(src3.at[wid], idx_s)
    lanes = _lanes()

    def fg(j, sc_x, sem):
        pltpu.async_copy(scores3.at[wid, j], sc_x.at[pl.ds(0, 4 * CH)], sem)

    def wg(sc_x, sem):
        pltpu.make_async_copy(scores3.at[wid, 0], sc_x.at[pl.ds(0, 4 * CH)],
                              sem).wait()

    def compute(j, sc_x, e_x, osem, guard):
        def _drain():
            pltpu.make_async_copy(e_x.at[pl.ds(0, 4 * CH)], e_o.at[wid, 0],
                                  osem).wait()

        if guard is True:
            _drain()
        else:
            pl.when(guard)(_drain)

        @pl.loop(0, CH // 16)
        def _group(g):
            srcv = idx_s[j, pl.ds(g * 16, 16)]
            rowi4 = (g * 16 + lanes) * 4
            for h in range(4):
                scv = plsc.load_gather(sc_x, [rowi4 + h])
                mg = plsc.load_gather(m_tab, [srcv * 4 + h])
                plsc.store_scatter(e_x, [rowi4 + h], jnp.exp(scv - mg))
            mk = lanes < 4
            off = jnp.where(mk, lanes, 0)
            for l in range(16):
                srce = srcv[l]
                e4 = e_x[pl.ds((g * 16 + l) * 4, 16)]
                sidx = jnp.where(mk, srce * 4 + off, 0)
                cur = plsc.load_gather(s_loc, [sidx], mask=mk)
                plsc.store_scatter(s_loc, [sidx], cur + e4, mask=mk)

        pltpu.async_copy(e_x.at[pl.ds(0, 4 * CH)], e_o.at[wid, j], osem)

    fg(0, sc_a, ga)

    @pl.loop(0, (NCHUNK - 1) // 2)
    def _pair(k):
        j = k * 2
        wg(sc_a, ga)
        fg(j + 1, sc_b, gb)
        compute(j, sc_a, e_va, oa, k > 0)
        wg(sc_b, gb)
        fg(j + 2, sc_a, ga)
        compute(j + 1, sc_b, e_vb, ob, k > 0)

    wg(sc_a, ga)
    compute(NCHUNK - 1, sc_a, e_va, oa, True)
    pltpu.make_async_copy(e_va.at[pl.ds(0, 4 * CH)], e_o.at[wid, 0],
                          oa).wait()
    pltpu.make_async_copy(e_vb.at[pl.ds(0, 4 * CH)], e_o.at[wid, 0],
                          ob).wait()
    pltpu.sync_copy(s_loc, spart_o.at[wid])


@functools.partial(
    pl.kernel,
    out_type=jax.ShapeDtypeStruct((N_EDGES,), F32),
    mesh=plsc.VectorSubcoreMesh(**_MESH),
    compiler_params=_SC_PARAMS,
    scratch_types=[
        pltpu.VMEM((NCHUNK, CH), I32),
        pltpu.VMEM((NCHUNK, CH), I32),
        pltpu.VMEM((CH, 256), F32),
        pltpu.VMEM((CH, 256), F32),
        pltpu.VMEM((CH, 256), F32),
        pltpu.VMEM((CH, 256), F32),
        pltpu.VMEM((CH, 4), F32),
        pltpu.VMEM((CH, 4), F32),
        pltpu.VMEM((4 * CH,), F32),
        pltpu.VMEM((4 * CH,), F32),
        pltpu.VMEM((CH,), F32),
        pltpu.VMEM((CH,), F32),
        pltpu.VMEM((CH,), F32),
        pltpu.VMEM((CH,), F32),
        pltpu.VMEM((256,), F32),
        pltpu.VMEM((256,), F32),
        pltpu.VMEM((256,), F32),
        pltpu.VMEM((16,), F32),
        pltpu.SemaphoreType.DMA,
        pltpu.SemaphoreType.DMA,
        pltpu.SemaphoreType.DMA,
        pltpu.SemaphoreType.DMA,
    ],
)
def _sc_att3(pf, qf, e3, src3, dst3, attr, recip2, fr, cvec, w2, b2, z_o,
             idx_s, idx_d, pf_a, qf_a, pf_b, qf_b, rc_a, rc_b, e_a, e_b,
             at_a, at_b, z_a, z_b, fr_v, cv_v, w2_v, b2_v, ga, gb, oa, ob):
    wid = _wid()
    pltpu.sync_copy(fr, fr_v)
    pltpu.sync_copy(cvec, cv_v)
    pltpu.sync_copy(w2, w2_v)
    pltpu.sync_copy(b2, b2_v)
    pltpu.sync_copy(src3.at[wid], idx_s)
    pltpu.sync_copy(dst3.at[wid], idx_d)
    fr_s = [fr_v[pl.ds(jj * 16, 16)] for jj in range(16)]
    cv_s = [cv_v[pl.ds(jj * 16, 16)] for jj in range(16)]
    w2_s = [w2_v[pl.ds(jj * 16, 16)] for jj in range(16)]
    b2_s = b2_v[pl.ds(0, 16)][0]
    lanes = _lanes()

    def fg(j, pf_x, qf_x, rc_x, e_x, at_x, sem):
        pltpu.async_copy(pf.at[idx_s.at[j]], pf_x, sem)
        pltpu.async_copy(qf.at[idx_d.at[j]], qf_x, sem)
        pltpu.async_copy(recip2.at[idx_s.at[j]], rc_x, sem)
        pltpu.async_copy(e3.at[wid, j], e_x, sem)
        pltpu.async_copy(attr.at[pl.ds(wid * EPW + j * CH, CH)], at_x, sem)

    def wg(pf_x, qf_x, rc_x, e_x, at_x, sem):
        pltpu.make_async_copy(pf.at[idx_s.at[0]], pf_x, sem).wait()
        pltpu.make_async_copy(qf.at[idx_d.at[0]], qf_x, sem).wait()
        pltpu.make_async_copy(recip2.at[idx_s.at[0]], rc_x, sem).wait()
        pltpu.make_async_copy(e3.at[wid, 0], e_x, sem).wait()
        pltpu.make_async_copy(attr.at[pl.ds(0, CH)], at_x, sem).wait()

    def compute(j, pf_x, qf_x, rc_x, e_x, at_x, z_x, osem, guard):
        def _drain():
            pltpu.make_async_copy(z_x, z_o.at[pl.ds(0, CH)], osem).wait()

        if guard is True:
            _drain()
        else:
            pl.when(guard)(_drain)

        @pl.loop(0, CH // 16)
        def _group(g):
            rowi = g * 16 + lanes
            rowi4 = rowi * 4
            att_g = jnp.zeros((16,), F32)
            for h in range(4):
                colh = jnp.zeros((16,), I32) + h
                rv = plsc.load_gather(rc_x, [rowi, colh])
                ev = plsc.load_gather(e_x, [rowi4 + h])
                att_g = att_g + ev * rv
            attr_g = at_x[pl.ds(g * 16, 16)]
            for l in range(16):
                e = g * 16 + l
                atte = att_g[l]
                attre = attr_g[l]
                acc = jnp.zeros((16,), F32)
                for jj in range(16):
                    pfv = pf_x[e, pl.ds(jj * 16, 16)]
                    qfv = qf_x[e, pl.ds(jj * 16, 16)]
                    hv = jnp.maximum(
                        atte * (pfv + qfv) + attre * fr_s[jj] + cv_s[jj], 0.0)
                    acc = acc + hv * w2_s[jj]
                _store1(z_x, e, jnp.sum(acc) + b2_s)

        pltpu.async_copy(z_x, z_o.at[pl.ds(wid * EPW + j * CH, CH)], osem)

    fg(0, pf_a, qf_a, rc_a, e_a, at_a, ga)

    @pl.loop(0, (NCHUNK - 1) // 2)
    def _pair(k):
        j = k * 2
        wg(pf_a, qf_a, rc_a, e_a, at_a, ga)
        fg(j + 1, pf_b, qf_b, rc_b, e_b, at_b, gb)
        compute(j, pf_a, qf_a, rc_a, e_a, at_a, z_a, oa, k > 0)
        wg(pf_b, qf_b, rc_b, e_b, at_b, gb)
        fg(j + 2, pf_a, qf_a, rc_a, e_a, at_a, ga)
        compute(j + 1, pf_b, qf_b, rc_b, e_b, at_b, z_b, ob, k > 0)

    wg(pf_a, qf_a, rc_a, e_a, at_a, ga)
    compute(NCHUNK - 1, pf_a, qf_a, rc_a, e_a, at_a, z_a, oa, True)
    pltpu.make_async_copy(z_a, z_o.at[pl.ds(0, CH)], oa).wait()
    pltpu.make_async_copy(z_b, z_o.at[pl.ds(0, CH)], ob).wait()


def _dis_col(cnt_t):
    return lax.rsqrt(jnp.sum(cnt_t, axis=1, keepdims=True) + 1.0)


def _tc_first(cnt_t_ref, x_ref, w_ref, out_ref):
    dis = _dis_col(cnt_t_ref[...])
    xw = jnp.dot(x_ref[...], w_ref[...], preferred_element_type=F32)
    out_ref[...] = xw * dis


def _tc_layer(cnt_t_ref, parts_ref, xws_ref, b_ref, w_ref, out_ref):
    dis = _dis_col(cnt_t_ref[...])
    acc = parts_ref[0] + parts_ref[1] + xws_ref[...]
    h = jnp.maximum(acc * dis + b_ref[...][None, :], 0.0)
    out_ref[...] = jnp.dot(h, w_ref[...], preferred_element_type=F32) * dis


def _tc_head(cnt_t_ref, parts_ref, xws_ref, b_ref, wa1_ref, ab_ref, wa2_ref,
             f1_ref, f2_ref, pa_ref, qa_ref, pf_ref, qf_ref):
    dis = _dis_col(cnt_t_ref[...])
    acc = parts_ref[0] + parts_ref[1] + xws_ref[...]
    h = jnp.maximum(acc * dis + b_ref[...][None, :], 0.0)
    pa_ref[...] = (jnp.dot(h, wa1_ref[...], preferred_element_type=F32)
                   + ab_ref[...][None, :])
    qa_ref[...] = jnp.dot(h, wa2_ref[...], preferred_element_type=F32)
    pf_ref[...] = jnp.dot(h, f1_ref[...], preferred_element_type=F32)
    qf_ref[...] = jnp.dot(h, f2_ref[...], preferred_element_type=F32)


def _tc_maxred(parts_ref, out_ref):
    out_ref[...] = jnp.max(parts_ref[...], axis=0, keepdims=True)


def _tc_recip(parts_ref, alpha_ref, out_ref):
    s = jnp.sum(parts_ref[...], axis=0, keepdims=True)
    out_ref[...] = (1.0 + alpha_ref[0, 0]) * 0.25 / (s + 1e-16)


def _tc_prob(z_ref, mu_ref, out_ref):
    z = z_ref[...]
    sp = jnp.maximum(z, 0.0) + jnp.log1p(jnp.exp(-jnp.abs(z)))
    out_ref[...] = jnp.clip(sp / mu_ref[0, 0], 0.0, 1.0)


def _vspec():
    return pl.BlockSpec(memory_space=pltpu.ANY)


def kernel(x, edge_index, edge_attr, alpha, mu, W1, b1, W2, b2, W3, b3,
           att_W, att_b, att_a, fc1_W, fc1_b, fc2_W, fc2_b):
    f32 = jnp.float32
    src3 = edge_index[0].reshape(NW, NCHUNK, CH)
    dst3 = edge_index[1].reshape(NW, NCHUNK, CH)
    attr = edge_attr.reshape(-1)

    zeros_n = jnp.zeros((N_NODES,), f32)
    zeros128 = jnp.zeros((N_NODES, D), f32)
    zeros4 = jnp.zeros((4 * N_NODES,), f32)
    neg4 = jnp.full((4 * N_NODES,), -1e30, f32)

    cntp = _sc_degree(dst3, zeros_n)
    cnt_t = cntp.T

    xw1s = pl.pallas_call(
        _tc_first,
        out_shape=jax.ShapeDtypeStruct((N_NODES, D), f32),
    )(cnt_t, x, W1)
    p1 = _sc_gcn(xw1s, src3, dst3, zeros128)

    xw2s = pl.pallas_call(
        _tc_layer,
        out_shape=jax.ShapeDtypeStruct((N_NODES, D), f32),
    )(cnt_t, p1, xw1s, b1, W2)
    p2 = _sc_gcn(xw2s, src3, dst3, zeros128)

    xw3s = pl.pallas_call(
        _tc_layer,
        out_shape=jax.ShapeDtypeStruct((N_NODES, D), f32),
    )(cnt_t, p2, xw2s, b2, W3)
    p3 = _sc_gcn(xw3s, src3, dst3, zeros128)

    pa, qa, pf, qf = pl.pallas_call(
        _tc_head,
        out_shape=(
            jax.ShapeDtypeStruct((N_NODES, 64), f32),
            jax.ShapeDtypeStruct((N_NODES, 64), f32),
            jax.ShapeDtypeStruct((N_NODES, 256), f32),
            jax.ShapeDtypeStruct((N_NODES, 256), f32),
        ),
    )(cnt_t, p3, xw3s, b3, att_W[:D], att_b, att_W[D:2 * D],
      fc1_W[:D], fc1_W[D:2 * D])

    aw0 = att_W[2 * D] * 20.0
    aa = att_a.reshape(-1)

    scores3, mpart = _sc_att1(pa, qa, src3, dst3, attr, aw0, aa, neg4)
    mtab = pl.pallas_call(
        _tc_maxred,
        out_shape=jax.ShapeDtypeStruct((1, 4 * N_NODES), f32),
    )(mpart).reshape(-1)

    e3, spart = _sc_att2(scores3, src3, mtab, zeros4)
    recip = pl.pallas_call(
        _tc_recip,
        out_shape=jax.ShapeDtypeStruct((1, 4 * N_NODES), f32),
        in_specs=[pl.BlockSpec(memory_space=pltpu.VMEM),
                  pl.BlockSpec(memory_space=pltpu.SMEM)],
    )(spart, alpha).reshape(N_NODES, 4)

    fr = fc1_W[2 * D]
    cvec = alpha[0, 0] * fc1_W[2 * D + 1] + mu[0, 0] * fc1_W[2 * D + 2] + fc1_b
    w2 = fc2_W.reshape(-1)
    b2 = jnp.pad(fc2_b, (0, 15))
    z = _sc_att3(pf, qf, e3, src3, dst3, attr, recip, fr, cvec, w2, b2)

    prob = pl.pallas_call(
        _tc_prob,
        out_shape=jax.ShapeDtypeStruct((N_EDGES // D, D), f32),
        in_specs=[pl.BlockSpec(memory_space=pltpu.VMEM),
                  pl.BlockSpec(memory_space=pltpu.SMEM)],
    )(z.reshape(N_EDGES // D, D), mu)
    return prob.reshape(N_EDGES, 1)
